# baseline (speedup 1.0000x reference)
import functools

import jax
import jax.numpy as jnp
from jax import lax
from jax.experimental import pallas as pl
from jax.experimental.pallas import tpu as pltpu
from jax.experimental.pallas import tpu_sc as plsc

_B = 4096
_L = 200
_EMB = 64
_TOTAL = _B * _L
_NW = 32
_PER_W = _TOTAL // _NW
_CHUNK = 800
_STEPS = _PER_W // _CHUNK
_PAIRS = _STEPS // 2

_mesh = plsc.VectorSubcoreMesh(core_axis_name="c", subcore_axis_name="s")


@functools.partial(
    pl.kernel,
    mesh=_mesh,
    out_type=jax.ShapeDtypeStruct((_TOTAL, _EMB), jnp.float32),
    scratch_types=[
        pltpu.VMEM((_STEPS, _CHUNK), jnp.int32),
        pltpu.VMEM((2, _CHUNK, _EMB), jnp.float32),
        pltpu.SemaphoreType.DMA,
    ],
    compiler_params=pltpu.CompilerParams(use_tc_tiling_on_sc=False),
)
def _emb_gather(idx_hbm, table_hbm, out_hbm, idx_v, rows_v, gsem):
    wid = lax.axis_index("s") * 2 + lax.axis_index("c")
    base = wid * _PER_W

    pltpu.sync_copy(idx_hbm.at[wid], idx_v)

    def _start(i, b):
        pltpu.async_copy(table_hbm.at[idx_v.at[i]], rows_v.at[b], gsem)

    def _finish(i, b):
        pltpu.make_async_copy(table_hbm.at[idx_v.at[i]], rows_v.at[b], gsem).wait()
        off = pl.multiple_of(base + i * _CHUNK, 8)
        pltpu.sync_copy(rows_v.at[b], out_hbm.at[pl.ds(off, _CHUNK)])

    _start(0, 0)

    def body(j, carry):
        i0 = 2 * j
        _start(i0 + 1, 1)
        _finish(i0, 0)

        @pl.when(j + 1 < _PAIRS)
        def _():
            _start(i0 + 2, 0)

        _finish(i0 + 1, 1)
        return carry

    lax.fori_loop(0, _PAIRS, body, 0)


def kernel(sequences, token_embedding_matrix):
    idx = sequences.reshape(_NW, _STEPS, _CHUNK).astype(jnp.int32)
    flat = _emb_gather(idx, token_embedding_matrix)
    return flat.reshape(_B, _L, _EMB), _L
